# baseline (speedup 1.0000x reference)
import functools

import jax
import jax.numpy as jnp
from jax.experimental import pallas as pl
from jax.experimental.pallas import tpu as pltpu

K = 3
EPS = 1e-06


def _body(pt_ref, pc_ref, it_ref, ic_ref,
          wi1_ref, bi1_ref, wi2_ref, bi2_ref,
          wg1a_ref, wg1b_ref, bg1_ref, wg2_ref, bg2_ref,
          wd1a_ref, wd1b_ref, bd1_ref, wd2_ref, bd2_ref,
          out_ref, feat_ref, *, n_img):
    j = pl.program_id(1)
    f32 = jnp.float32

    @pl.when(j == 0)
    def _():
        x = it_ref[0]
        h = jnp.maximum(jnp.dot(x, wi1_ref[...], preferred_element_type=f32)
                        + bi1_ref[...], 0.0)
        feat_ref[...] = (jnp.dot(h, wi2_ref[...], preferred_element_type=f32)
                         + bi2_ref[...])

    q = pc_ref[0]
    s = ic_ref[0]
    q_sq = jnp.sum(q * q, axis=1, keepdims=True)
    s_sq = jnp.sum(s * s, axis=1, keepdims=True)
    cross = jax.lax.dot_general(-2.0 * s, q, (((1,), (1,)), ((), ())),
                                preferred_element_type=f32)
    sqd = s_sq + q_sq.T + cross

    iota = jax.lax.broadcasted_iota(jnp.int32, sqd.shape, 0)
    dbits = jnp.maximum(jax.lax.bitcast_convert_type(sqd, jnp.int32), 0)
    key = jnp.bitwise_or(jnp.bitwise_and(dbits, jnp.int32(-n_img)), iota)
    big = jnp.int32(0x7FFFFFFF)

    mk1 = jnp.min(key, axis=0, keepdims=True)
    mk2 = jnp.min(jnp.where(key == mk1, big, key), axis=0, keepdims=True)
    mk3 = jnp.min(jnp.where((key == mk1) | (key == mk2), big, key),
                  axis=0, keepdims=True)

    unq = lambda mk: jax.lax.bitcast_convert_type(
        jnp.bitwise_and(mk, jnp.int32(-n_img)), f32)
    w1 = 1.0 / jnp.maximum(jnp.sqrt(unq(mk1)), EPS)
    w2 = 1.0 / jnp.maximum(jnp.sqrt(unq(mk2)), EPS)
    w3 = 1.0 / jnp.maximum(jnp.sqrt(unq(mk3)), EPS)
    inv = 1.0 / jnp.maximum(w1 + w2 + w3, EPS)
    comb = jnp.where(key == mk1, w1 * inv,
                     jnp.where(key == mk2, w2 * inv,
                               jnp.where(key == mk3, w3 * inv, 0.0)))

    aligned = jax.lax.dot_general(comb, feat_ref[...],
                                  (((0,), (0,)), ((), ())),
                                  preferred_element_type=f32)

    point = pt_ref[0]

    hg = jnp.maximum(jnp.dot(point, wg1a_ref[...], preferred_element_type=f32)
                     + jnp.dot(aligned, wg1b_ref[...], preferred_element_type=f32)
                     + bg1_ref[...], 0.0)
    gate = jax.nn.sigmoid(jnp.dot(hg, wg2_ref[...], preferred_element_type=f32)
                          + bg2_ref[...])
    hd_ = jnp.maximum(jnp.dot(point, wd1a_ref[...], preferred_element_type=f32)
                      + jnp.dot(aligned, wd1b_ref[...], preferred_element_type=f32)
                      + bd1_ref[...], 0.0)
    delta = (jnp.dot(hd_, wd2_ref[...], preferred_element_type=f32)
             + bd2_ref[...])

    out_ref[0] = point + gate * delta


def kernel(point_token, patch_center, image_patch_token, image_patch_coord,
           Wi1, bi1, Wi2, bi2, Wg1, bg1, Wg2, bg2, Wd1, bd1, Wd2, bd2):
    B, Np, od = point_token.shape
    Ni, idim = image_patch_token.shape[1:]
    hd = Wi1.shape[1]
    BN = min(2048, Np)

    b2 = lambda b: b.reshape(1, -1)

    full = lambda arr: pl.BlockSpec(arr.shape, lambda b, j: (0,) * arr.ndim)
    grid = (B, Np // BN)

    out = pl.pallas_call(
        functools.partial(_body, n_img=Ni),
        grid=grid,
        in_specs=[
            pl.BlockSpec((1, BN, od), lambda b, j: (b, j, 0)),
            pl.BlockSpec((1, BN, 3), lambda b, j: (b, j, 0)),
            pl.BlockSpec((1, Ni, idim), lambda b, j: (b, 0, 0)),
            pl.BlockSpec((1, Ni, 3), lambda b, j: (b, 0, 0)),
            full(Wi1), pl.BlockSpec((1, hd), lambda b, j: (0, 0)),
            full(Wi2), pl.BlockSpec((1, od), lambda b, j: (0, 0)),
            pl.BlockSpec((od, hd), lambda b, j: (0, 0)),
            pl.BlockSpec((od, hd), lambda b, j: (0, 0)),
            pl.BlockSpec((1, hd), lambda b, j: (0, 0)),
            full(Wg2), pl.BlockSpec((1, od), lambda b, j: (0, 0)),
            pl.BlockSpec((od, hd), lambda b, j: (0, 0)),
            pl.BlockSpec((od, hd), lambda b, j: (0, 0)),
            pl.BlockSpec((1, hd), lambda b, j: (0, 0)),
            full(Wd2), pl.BlockSpec((1, od), lambda b, j: (0, 0)),
        ],
        out_specs=pl.BlockSpec((1, BN, od), lambda b, j: (b, j, 0)),
        out_shape=jax.ShapeDtypeStruct((B, Np, od), jnp.float32),
        scratch_shapes=[pltpu.VMEM((Ni, od), jnp.float32)],
        compiler_params=pltpu.CompilerParams(
            dimension_semantics=("arbitrary", "arbitrary")),
    )(point_token, patch_center, image_patch_token, image_patch_coord,
      Wi1, b2(bi1), Wi2, b2(bi2),
      Wg1[:od], Wg1[od:], b2(bg1), Wg2, b2(bg2),
      Wd1[:od], Wd1[od:], b2(bd1), Wd2, b2(bd2))
    return out


# speedup vs baseline: 1.0705x; 1.0705x over previous
import functools

import jax
import jax.numpy as jnp
from jax.experimental import pallas as pl
from jax.experimental.pallas import tpu as pltpu

K = 3
EPS = 1e-06


def _body(pt_ref, pc_ref, it_ref, ic_ref,
          wi1_ref, bi1_ref, wi2_ref, bi2_ref,
          wg1_ref, bg1_ref, wg2_ref, bg2_ref,
          wd1_ref, bd1_ref, wd2_ref, bd2_ref,
          out_ref, feat_ref, *, n_img):
    j = pl.program_id(1)
    f32 = jnp.float32

    @pl.when(j == 0)
    def _():
        x = it_ref[0]
        h = jnp.maximum(jnp.dot(x, wi1_ref[...], preferred_element_type=f32)
                        + bi1_ref[...], 0.0)
        feat_ref[...] = (jnp.dot(h, wi2_ref[...], preferred_element_type=f32)
                         + bi2_ref[...])

    q = pc_ref[0]
    s = ic_ref[0]
    q_sq = jnp.sum(q * q, axis=1, keepdims=True)
    s_sq = jnp.sum(s * s, axis=1, keepdims=True)
    cross = jax.lax.dot_general(-2.0 * s, q, (((1,), (1,)), ((), ())),
                                preferred_element_type=f32)
    sqd = s_sq + q_sq.T + cross

    iota = jax.lax.broadcasted_iota(jnp.int32, sqd.shape, 0)
    dbits = jnp.maximum(jax.lax.bitcast_convert_type(sqd, jnp.int32), 0)
    key = jnp.bitwise_or(jnp.bitwise_and(dbits, jnp.int32(-n_img)), iota)
    big = jnp.int32(0x7FFFFFFF)

    mk1 = jnp.min(key, axis=0, keepdims=True)
    mk2 = jnp.min(jnp.where(key == mk1, big, key), axis=0, keepdims=True)
    mk3 = jnp.min(jnp.where((key == mk1) | (key == mk2), big, key),
                  axis=0, keepdims=True)

    unq = lambda mk: jax.lax.bitcast_convert_type(
        jnp.bitwise_and(mk, jnp.int32(-n_img)), f32)
    w1 = 1.0 / jnp.maximum(jnp.sqrt(unq(mk1)), EPS)
    w2 = 1.0 / jnp.maximum(jnp.sqrt(unq(mk2)), EPS)
    w3 = 1.0 / jnp.maximum(jnp.sqrt(unq(mk3)), EPS)
    inv = 1.0 / jnp.maximum(w1 + w2 + w3, EPS)
    comb = jnp.where(key == mk1, w1 * inv,
                     jnp.where(key == mk2, w2 * inv,
                               jnp.where(key == mk3, w3 * inv, 0.0)))

    aligned = jax.lax.dot_general(comb, feat_ref[...],
                                  (((0,), (0,)), ((), ())),
                                  preferred_element_type=f32)

    point = pt_ref[0]
    x = jnp.concatenate([point, aligned], axis=1)

    hg = jnp.maximum(jnp.dot(x, wg1_ref[...], preferred_element_type=f32)
                     + bg1_ref[...], 0.0)
    gate = jax.nn.sigmoid(jnp.dot(hg, wg2_ref[...], preferred_element_type=f32)
                          + bg2_ref[...])
    hd_ = jnp.maximum(jnp.dot(x, wd1_ref[...], preferred_element_type=f32)
                      + bd1_ref[...], 0.0)
    delta = (jnp.dot(hd_, wd2_ref[...], preferred_element_type=f32)
             + bd2_ref[...])

    out_ref[0] = point + gate * delta


def kernel(point_token, patch_center, image_patch_token, image_patch_coord,
           Wi1, bi1, Wi2, bi2, Wg1, bg1, Wg2, bg2, Wd1, bd1, Wd2, bd2):
    B, Np, od = point_token.shape
    Ni, idim = image_patch_token.shape[1:]
    hd = Wi1.shape[1]
    BN = min(2048, Np)

    b2 = lambda b: b.reshape(1, -1)

    full = lambda arr: pl.BlockSpec(arr.shape, lambda b, j: (0,) * arr.ndim)
    grid = (B, Np // BN)

    out = pl.pallas_call(
        functools.partial(_body, n_img=Ni),
        grid=grid,
        in_specs=[
            pl.BlockSpec((1, BN, od), lambda b, j: (b, j, 0)),
            pl.BlockSpec((1, BN, 3), lambda b, j: (b, j, 0)),
            pl.BlockSpec((1, Ni, idim), lambda b, j: (b, 0, 0)),
            pl.BlockSpec((1, Ni, 3), lambda b, j: (b, 0, 0)),
            full(Wi1), pl.BlockSpec((1, hd), lambda b, j: (0, 0)),
            full(Wi2), pl.BlockSpec((1, od), lambda b, j: (0, 0)),
            full(Wg1), pl.BlockSpec((1, hd), lambda b, j: (0, 0)),
            full(Wg2), pl.BlockSpec((1, od), lambda b, j: (0, 0)),
            full(Wd1), pl.BlockSpec((1, hd), lambda b, j: (0, 0)),
            full(Wd2), pl.BlockSpec((1, od), lambda b, j: (0, 0)),
        ],
        out_specs=pl.BlockSpec((1, BN, od), lambda b, j: (b, j, 0)),
        out_shape=jax.ShapeDtypeStruct((B, Np, od), jnp.float32),
        scratch_shapes=[pltpu.VMEM((Ni, od), jnp.float32)],
        compiler_params=pltpu.CompilerParams(
            dimension_semantics=("arbitrary", "arbitrary")),
    )(point_token, patch_center, image_patch_token, image_patch_coord,
      Wi1, b2(bi1), Wi2, b2(bi2), Wg1, b2(bg1), Wg2, b2(bg2),
      Wd1, b2(bd1), Wd2, b2(bd2))
    return out
